# Initial kernel scaffold; baseline (speedup 1.0000x reference)
import jax
import jax.numpy as jnp
from jax.experimental import pallas as pl


def kernel(score, target):
    raise NotImplementedError("write your pallas kernel here")


# speedup vs baseline: 73.2607x; 73.2607x over previous
import functools

import jax
import jax.numpy as jnp
from jax import lax
from jax.experimental import pallas as pl
from jax.experimental.pallas import tpu as pltpu
from jax.experimental.pallas import tpu_sc as plsc

_N = 16 * 512 * 512
_NC = 2
_NS = 16
_NW = _NC * _NS
_PER_W = _N // _NW
_CHUNK = 16384
_NCHUNK = _PER_W // _CHUNK
_LANES = 16
_LOG_THRESH = 0.6931471805599453


def _sc_partials(score_flat, target_flat):
    mesh = plsc.VectorSubcoreMesh(core_axis_name="c", subcore_axis_name="s")

    @functools.partial(
        pl.kernel,
        mesh=mesh,
        out_type=[
            jax.ShapeDtypeStruct((_NW * _LANES,), jnp.float32),
            jax.ShapeDtypeStruct((_NW * _LANES,), jnp.float32),
        ],
        scratch_types=[
            pltpu.VMEM((2, _CHUNK), jnp.float32),
            pltpu.VMEM((2, _CHUNK), jnp.float32),
            pltpu.VMEM((_LANES,), jnp.float32),
            pltpu.VMEM((_LANES,), jnp.float32),
            pltpu.SemaphoreType.DMA,
        ],
    )
    def k(score_hbm, target_hbm, sums_hbm, cnts_hbm,
          sbuf, tbuf, svec, cvec, sem):
        wid = lax.axis_index("s") * _NC + lax.axis_index("c")
        base = wid * _PER_W

        def start(ci, slot):
            off = pl.multiple_of(base + ci * _CHUNK, _CHUNK)
            a = pltpu.make_async_copy(
                score_hbm.at[pl.ds(off, _CHUNK)], sbuf.at[slot], sem)
            b = pltpu.make_async_copy(
                target_hbm.at[pl.ds(off, _CHUNK)], tbuf.at[slot], sem)
            a.start()
            b.start()
            return a, b

        inflight = [None, None]
        inflight[0] = start(0, 0)

        zero = jnp.zeros((_LANES,), jnp.float32)
        acc, cnt = zero, zero
        for ci in range(_NCHUNK):
            slot = ci % 2
            inflight[slot][0].wait()
            inflight[slot][1].wait()
            if ci + 1 < _NCHUNK:
                inflight[1 - slot] = start(ci + 1, 1 - slot)

            def body(i, c2, slot=slot):
                a, n = c2
                s = sbuf[slot, pl.ds(i * _LANES, _LANES)]
                t = tbuf[slot, pl.ds(i * _LANES, _LANES)]
                lp = s - s
                m = lp < _LOG_THRESH
                a = a - jnp.where(m, t * lp, 0.0)
                n = n + jnp.where(m, 1.0, 0.0)
                return a, n

            acc, cnt = lax.fori_loop(0, _CHUNK // _LANES, body, (acc, cnt))

        svec[...] = acc
        cvec[...] = cnt
        out_off = pl.multiple_of(wid * _LANES, 8)
        pltpu.sync_copy(svec, sums_hbm.at[pl.ds(out_off, _LANES)])
        pltpu.sync_copy(cvec, cnts_hbm.at[pl.ds(out_off, _LANES)])

    return k(score_flat, target_flat)


def _tc_finish(sums, cnts):
    def body(s_ref, c_ref, o_ref):
        o_ref[...] = jnp.broadcast_to(
            jnp.sum(s_ref[...]) / jnp.sum(c_ref[...]), (1, 1))

    out = pl.pallas_call(
        body,
        out_shape=jax.ShapeDtypeStruct((1, 1), jnp.float32),
    )(sums.reshape(4, 128), cnts.reshape(4, 128))
    return out[0, 0]


def kernel(score, target):
    sums, cnts = _sc_partials(score.reshape(-1), target.reshape(-1))
    return _tc_finish(sums, cnts)


# speedup vs baseline: 166.2810x; 2.2697x over previous
import functools

import jax
import jax.numpy as jnp
from jax import lax
from jax.experimental import pallas as pl
from jax.experimental.pallas import tpu as pltpu
from jax.experimental.pallas import tpu_sc as plsc

_N = 16 * 512 * 512
_W = 512
_ROWS = _N // _W
_NC = 2
_NS = 16
_NW = _NC * _NS
_ROWS_W = _ROWS // _NW
_CROWS = 32
_NCHUNK = _ROWS_W // _CROWS
_LANES = 16
_VPR = _W // _LANES
_LOG_THRESH = 0.6931471805599453


def _sc_partials(score2d, target2d):
    mesh = plsc.VectorSubcoreMesh(core_axis_name="c", subcore_axis_name="s")

    @functools.partial(
        pl.kernel,
        mesh=mesh,
        out_type=[
            jax.ShapeDtypeStruct((_NW * _LANES,), jnp.float32),
            jax.ShapeDtypeStruct((_NW * _LANES,), jnp.float32),
        ],
        scratch_types=[
            pltpu.VMEM((2, _CROWS, _W), jnp.float32),
            pltpu.VMEM((2, _CROWS, _W), jnp.float32),
            pltpu.VMEM((_LANES,), jnp.float32),
            pltpu.VMEM((_LANES,), jnp.float32),
            pltpu.SemaphoreType.DMA,
        ],
    )
    def k(score_hbm, target_hbm, sums_hbm, cnts_hbm,
          sbuf, tbuf, svec, cvec, sem):
        wid = lax.axis_index("s") * _NC + lax.axis_index("c")
        base = wid * _ROWS_W

        def start(ci, slot):
            row = pl.multiple_of(base + ci * _CROWS, _CROWS)
            a = pltpu.make_async_copy(
                score_hbm.at[pl.ds(row, _CROWS), :], sbuf.at[slot], sem)
            b = pltpu.make_async_copy(
                target_hbm.at[pl.ds(row, _CROWS), :], tbuf.at[slot], sem)
            a.start()
            b.start()
            return a, b

        inflight = [None, None]
        inflight[0] = start(0, 0)

        zero = jnp.zeros((_LANES,), jnp.float32)
        acc, cnt = zero, zero
        for ci in range(_NCHUNK):
            slot = ci % 2
            inflight[slot][0].wait()
            inflight[slot][1].wait()
            if ci + 1 < _NCHUNK:
                inflight[1 - slot] = start(ci + 1, 1 - slot)

            def row_body(r, c2, slot=slot):
                a, n = c2
                for j in range(_VPR):
                    s = sbuf[slot, r, j * _LANES:(j + 1) * _LANES]
                    t = tbuf[slot, r, j * _LANES:(j + 1) * _LANES]
                    lp = s - s
                    m = lp < _LOG_THRESH
                    a = a - jnp.where(m, t * lp, 0.0)
                    n = n + jnp.where(m, 1.0, 0.0)
                return a, n

            acc, cnt = lax.fori_loop(0, _CROWS, row_body, (acc, cnt))

        svec[...] = acc
        cvec[...] = cnt
        out_off = pl.multiple_of(wid * _LANES, 8)
        pltpu.sync_copy(svec, sums_hbm.at[pl.ds(out_off, _LANES)])
        pltpu.sync_copy(cvec, cnts_hbm.at[pl.ds(out_off, _LANES)])

    return k(score2d, target2d)


def _tc_finish(sums, cnts):
    def body(s_ref, c_ref, o_ref):
        o_ref[...] = jnp.broadcast_to(
            jnp.sum(s_ref[...]) / jnp.sum(c_ref[...]), (1, 1))

    out = pl.pallas_call(
        body,
        out_shape=jax.ShapeDtypeStruct((1, 1), jnp.float32),
    )(sums.reshape(4, 128), cnts.reshape(4, 128))
    return out[0, 0]


def kernel(score, target):
    sums, cnts = _sc_partials(score.reshape(_ROWS, _W),
                              target.reshape(_ROWS, _W))
    return _tc_finish(sums, cnts)


# speedup vs baseline: 201.9297x; 1.2144x over previous
import functools

import jax
import jax.numpy as jnp
from jax import lax
from jax.experimental import pallas as pl
from jax.experimental.pallas import tpu as pltpu
from jax.experimental.pallas import tpu_sc as plsc

_N = 16 * 512 * 512
_W = 512
_ROWS = _N // _W
_NC = 2
_NS = 16
_NW = _NC * _NS
_SC_ROWS = 2048
_ROWS_W = _SC_ROWS // _NW
_CROWS = 32
_NCHUNK = _ROWS_W // _CROWS
_LANES = 16
_VPR = _W // _LANES
_TC_BLOCK = 512
_LOG_THRESH = 0.6931471805599453


def _sc_partials(score2d, target2d):
    mesh = plsc.VectorSubcoreMesh(core_axis_name="c", subcore_axis_name="s")

    @functools.partial(
        pl.kernel,
        mesh=mesh,
        out_type=[
            jax.ShapeDtypeStruct((_NW * _LANES,), jnp.float32),
            jax.ShapeDtypeStruct((_NW * _LANES,), jnp.float32),
        ],
        scratch_types=[
            pltpu.VMEM((2, _CROWS, _W), jnp.float32),
            pltpu.VMEM((2, _CROWS, _W), jnp.float32),
            pltpu.VMEM((_LANES,), jnp.float32),
            pltpu.VMEM((_LANES,), jnp.float32),
            pltpu.SemaphoreType.DMA,
        ],
    )
    def k(score_hbm, target_hbm, sums_hbm, cnts_hbm,
          sbuf, tbuf, svec, cvec, sem):
        wid = lax.axis_index("s") * _NC + lax.axis_index("c")
        base = wid * _ROWS_W

        def start(ci, slot):
            row = pl.multiple_of(base + ci * _CROWS, _CROWS)
            a = pltpu.make_async_copy(
                score_hbm.at[pl.ds(row, _CROWS), :], sbuf.at[slot], sem)
            b = pltpu.make_async_copy(
                target_hbm.at[pl.ds(row, _CROWS), :], tbuf.at[slot], sem)
            a.start()
            b.start()
            return a, b

        inflight = [None, None]
        inflight[0] = start(0, 0)

        zero = jnp.zeros((_LANES,), jnp.float32)
        acc, cnt = zero, zero
        for ci in range(_NCHUNK):
            slot = ci % 2
            inflight[slot][0].wait()
            inflight[slot][1].wait()
            if ci + 1 < _NCHUNK:
                inflight[1 - slot] = start(ci + 1, 1 - slot)

            def row_body(r, c2, slot=slot):
                a, n = c2
                for j in range(_VPR):
                    s = sbuf[slot, r, j * _LANES:(j + 1) * _LANES]
                    t = tbuf[slot, r, j * _LANES:(j + 1) * _LANES]
                    lp = s - s
                    m = lp < _LOG_THRESH
                    a = a - jnp.where(m, t * lp, 0.0)
                    n = n + jnp.where(m, 1.0, 0.0)
                return a, n

            acc, cnt = lax.fori_loop(0, _CROWS, row_body, (acc, cnt))

        svec[...] = acc
        cvec[...] = cnt
        out_off = pl.multiple_of(wid * _LANES, 8)
        pltpu.sync_copy(svec, sums_hbm.at[pl.ds(out_off, _LANES)])
        pltpu.sync_copy(cvec, cnts_hbm.at[pl.ds(out_off, _LANES)])

    return k(score2d, target2d)


def _tc_partials(score2d, target2d):
    n_blocks = (_ROWS - _SC_ROWS) // _TC_BLOCK

    def body(s_ref, t_ref, sum_ref, cnt_ref):
        @pl.when(pl.program_id(0) == 0)
        def _():
            sum_ref[...] = jnp.zeros_like(sum_ref)
            cnt_ref[...] = jnp.zeros_like(cnt_ref)

        s = s_ref[...]
        t = t_ref[...]
        lp = s - s
        m = lp < _LOG_THRESH
        loss = jnp.where(m, t * lp, 0.0)
        sum_ref[...] += jnp.broadcast_to(-jnp.sum(loss), (1, 1))
        cnt_ref[...] += jnp.broadcast_to(
            jnp.sum(m.astype(jnp.float32)), (1, 1))

    in_spec = pl.BlockSpec(
        (_TC_BLOCK, _W), lambda i: (i + _SC_ROWS // _TC_BLOCK, 0))
    out_spec = pl.BlockSpec((1, 1), lambda i: (0, 0))
    return pl.pallas_call(
        body,
        grid=(n_blocks,),
        in_specs=[in_spec, in_spec],
        out_specs=[out_spec, out_spec],
        out_shape=[jax.ShapeDtypeStruct((1, 1), jnp.float32),
                   jax.ShapeDtypeStruct((1, 1), jnp.float32)],
    )(score2d, target2d)


def _combine(sc_sums, sc_cnts, tc_sum, tc_cnt):
    def body(ss_ref, sc_ref, ts_ref, tn_ref, o_ref):
        total = jnp.sum(ss_ref[...]) + ts_ref[0, 0]
        count = jnp.sum(sc_ref[...]) + tn_ref[0, 0]
        o_ref[...] = jnp.broadcast_to(total / count, (1, 1))

    out = pl.pallas_call(
        body,
        out_shape=jax.ShapeDtypeStruct((1, 1), jnp.float32),
    )(sc_sums.reshape(4, 128), sc_cnts.reshape(4, 128), tc_sum, tc_cnt)
    return out[0, 0]


def kernel(score, target):
    score2d = score.reshape(_ROWS, _W)
    target2d = target.reshape(_ROWS, _W)
    sc_sums, sc_cnts = _sc_partials(score2d, target2d)
    tc_sum, tc_cnt = _tc_partials(score2d, target2d)
    return _combine(sc_sums, sc_cnts, tc_sum, tc_cnt)
